# baseline (speedup 1.0000x reference)
import functools

import jax
import jax.numpy as jnp
from jax import lax
from jax.experimental import pallas as pl
from jax.experimental.pallas import tpu as pltpu
from jax.experimental.pallas import tpu_sc as plsc

VOCAB = 128256
D = 496
T_FAST = 8
T_SLOW = 2
T_ALL = T_FAST + T_SLOW

RT = D // 8
CT = VOCAB // 128

NC = 2
NS = 16
NW = NC * NS
G = 16


def _sc_gather_cols(ids, z, n):
    per = n // NW
    mesh = plsc.VectorSubcoreMesh(core_axis_name="c", subcore_axis_name="s")
    nslot = 4

    @functools.partial(
        pl.kernel,
        mesh=mesh,
        compiler_params=pltpu.CompilerParams(
            use_tc_tiling_on_sc=False, needs_layout_passes=False),
        out_type=jax.ShapeDtypeStruct((RT, n // 128, 8, 128), jnp.float32),
        scratch_types=[
            pltpu.VMEM((per,), jnp.int32),
            pltpu.VMEM((nslot, RT, 8, 8), jnp.float32),
            pltpu.VMEM((RT, 8, G), jnp.float32),
        ] + [pltpu.SemaphoreType.DMA] * nslot,
    )
    def k(ids_hbm, z_hbm, out_hbm, idx_v, bufn, slab, *sems):
        wid = lax.axis_index("s") * NC + lax.axis_index("c")
        base = wid * per
        pltpu.sync_copy(ids_hbm.at[pl.ds(base, per)], idx_v)
        iota = lax.iota(jnp.int32, G)

        def token_id(t):
            vec = idx_v[pl.ds((t // G) * G, G)]
            return jnp.sum(jnp.where(iota == t % G, vec, 0))

        def issue(t, buf, sem):
            v = token_id(t)
            c = v >> 7
            col8 = pl.multiple_of((v & 127) & ~7, 8)
            pltpu.async_copy(
                z_hbm.at[:, c, :, pl.ds(col8, 8)], buf, sem)

        def extract(t, buf, sem):
            pltpu.make_async_copy(z_hbm.at[:, 0, :, pl.ds(0, 8)], buf, sem
                                  ).wait()
            off = token_id(t) & 7
            offv = jnp.full((G,), 0, jnp.int32) + off
            jv = jnp.full((G,), 0, jnp.int32) + (t % G)
            for kk in range(RT * 8 // G):
                d = iota + kk * G
                a = d >> 3
                f = d & 7
                vals = plsc.load_gather(buf, [a, f, offv])
                plsc.store_scatter(slab, [a, f, jv], vals)

        def flush(t0):
            tok0 = base + (t0 // G) * G
            sub = pl.multiple_of(tok0 % 128, G)
            pltpu.sync_copy(
                slab, out_hbm.at[:, tok0 // 128, :, pl.ds(sub, G)]
            )

        for i in range(nslot):
            issue(i, bufn.at[i], sems[i])

        def body(q, carry):
            t = nslot * q
            for i in range(nslot):
                extract(t + i, bufn.at[i], sems[i])

                @pl.when(t + i + nslot < per)
                def _():
                    issue(t + i + nslot, bufn.at[i], sems[i])

            @pl.when((t + nslot) % G == 0)
            def _():
                flush(t)

            return carry

        lax.fori_loop(0, per // nslot, body, 0)

    return k(ids, z)


def _tc_expand_t(eT, W, fast_basis, slow_basis, drive_scale, slow_scale,
                 tile, n_total, blk0, prev=None):
    n = eT.shape[1]

    def body(w_ref, e_ref, fb_ref, sb_ref, ds_ref, ss_ref, *rest):
        out_ref = rest[-1]
        xT = lax.dot_general(
            w_ref[...], e_ref[...], (((1,), (0,)), ((), ())),
            preferred_element_type=jnp.float32,
        )
        cf = jax.nn.sigmoid(fb_ref[...]) * ds_ref[0]
        cs = jax.nn.sigmoid(sb_ref[...]) * ss_ref[0]
        coef = jnp.concatenate([cf, cs], axis=0)
        out_ref[...] = coef[:, :, None] * xT[None, :, :]

    in_specs = [
        pl.BlockSpec((D, D), lambda i: (0, 0)),
        pl.BlockSpec((D, tile), lambda i: (0, i)),
        pl.BlockSpec((T_FAST, D), lambda i: (0, 0)),
        pl.BlockSpec((T_SLOW, D), lambda i: (0, 0)),
        pl.BlockSpec(memory_space=pltpu.SMEM),
        pl.BlockSpec(memory_space=pltpu.SMEM),
    ]
    args = [W, eT, fast_basis, slow_basis,
            drive_scale.reshape(1), slow_scale.reshape(1)]
    aliases = {}
    if prev is not None:
        in_specs.append(pl.BlockSpec(memory_space=pl.ANY))
        args.append(prev)
        aliases = {6: 0}
    return pl.pallas_call(
        body,
        grid=(n // tile,),
        in_specs=in_specs,
        out_specs=pl.BlockSpec((T_ALL, D, tile),
                               lambda i: (0, 0, i + blk0)),
        out_shape=jax.ShapeDtypeStruct((T_ALL, D, n_total), jnp.float32),
        input_output_aliases=aliases,
    )(*args)


def kernel(token_ids, embed, W, drive_scale, fast_basis, slow_basis, slow_scale):
    n = token_ids.size
    ids = token_ids.reshape(-1).astype(jnp.int32)
    tT = jnp.swapaxes(embed, 0, 1)
    z = tT.reshape(RT, 8, CT, 128).transpose(0, 2, 1, 3)
    tile = 512
    n_chunk = 2
    nh = n // n_chunk
    outT = None
    for chunk in range(n_chunk):
        ids_h = lax.slice_in_dim(ids, chunk * nh, (chunk + 1) * nh)
        e4 = _sc_gather_cols(ids_h, z, nh)
        eT = e4.transpose(0, 2, 1, 3).reshape(D, nh)
        outT = _tc_expand_t(eT, W, fast_basis, slow_basis, drive_scale,
                            slow_scale, tile, n, chunk * (nh // tile),
                            prev=outT)
    return jnp.swapaxes(outT, 1, 2)


# speedup vs baseline: 1.0132x; 1.0132x over previous
import functools

import jax
import jax.numpy as jnp
from jax import lax
from jax.experimental import pallas as pl
from jax.experimental.pallas import tpu as pltpu
from jax.experimental.pallas import tpu_sc as plsc

VOCAB = 128256
D = 496
T_FAST = 8
T_SLOW = 2
T_ALL = T_FAST + T_SLOW

RT = D // 8
CT = VOCAB // 128

NC = 2
NS = 16
NW = NC * NS
G = 16


def _sc_gather_cols(ids, z, n):
    per = n // NW
    mesh = plsc.VectorSubcoreMesh(core_axis_name="c", subcore_axis_name="s")
    nslot = 4

    @functools.partial(
        pl.kernel,
        mesh=mesh,
        compiler_params=pltpu.CompilerParams(
            use_tc_tiling_on_sc=False, needs_layout_passes=False),
        out_type=jax.ShapeDtypeStruct((RT, n // 128, 8, 128), jnp.float32),
        scratch_types=[
            pltpu.VMEM((per,), jnp.int32),
            pltpu.VMEM((nslot, RT, 8, 8), jnp.float32),
            pltpu.VMEM((RT, 8, G), jnp.float32),
        ] + [pltpu.SemaphoreType.DMA] * nslot,
    )
    def k(ids_hbm, z_hbm, out_hbm, idx_v, bufn, slab, *sems):
        wid = lax.axis_index("s") * NC + lax.axis_index("c")
        base = wid * per
        pltpu.sync_copy(ids_hbm.at[pl.ds(base, per)], idx_v)
        iota = lax.iota(jnp.int32, G)

        def token_id(t):
            vec = idx_v[pl.ds((t // G) * G, G)]
            return jnp.sum(jnp.where(iota == t % G, vec, 0))

        def issue(t, buf, sem):
            v = token_id(t)
            c = v >> 7
            col8 = pl.multiple_of((v & 127) & ~7, 8)
            pltpu.async_copy(
                z_hbm.at[:, c, :, pl.ds(col8, 8)], buf, sem)

        def extract(t, buf, sem):
            pltpu.make_async_copy(z_hbm.at[:, 0, :, pl.ds(0, 8)], buf, sem
                                  ).wait()
            off = token_id(t) & 7
            offv = jnp.full((G,), 0, jnp.int32) + off
            jv = jnp.full((G,), 0, jnp.int32) + (t % G)
            for kk in range(RT * 8 // G):
                d = iota + kk * G
                a = d >> 3
                f = d & 7
                vals = plsc.load_gather(buf, [a, f, offv])
                plsc.store_scatter(slab, [a, f, jv], vals)

        def flush(t0):
            tok0 = base + (t0 // G) * G
            sub = pl.multiple_of(tok0 % 128, G)
            pltpu.sync_copy(
                slab, out_hbm.at[:, tok0 // 128, :, pl.ds(sub, G)]
            )

        for i in range(nslot):
            issue(i, bufn.at[i], sems[i])

        def body(q, carry):
            t = nslot * q
            for i in range(nslot):
                extract(t + i, bufn.at[i], sems[i])

                @pl.when(t + i + nslot < per)
                def _():
                    issue(t + i + nslot, bufn.at[i], sems[i])

            @pl.when((t + nslot) % G == 0)
            def _():
                flush(t)

            return carry

        lax.fori_loop(0, per // nslot, body, 0)

    return k(ids, z)


def _tc_expand_t(eT, W, fast_basis, slow_basis, drive_scale, slow_scale,
                 tile, n_total, blk0, prev=None):
    n = eT.shape[1]

    def body(w_ref, e_ref, fb_ref, sb_ref, ds_ref, ss_ref, *rest):
        out_ref = rest[-1]
        xT = lax.dot_general(
            w_ref[...], e_ref[...], (((1,), (0,)), ((), ())),
            preferred_element_type=jnp.float32,
        )
        cf = jax.nn.sigmoid(fb_ref[...]) * ds_ref[0]
        cs = jax.nn.sigmoid(sb_ref[...]) * ss_ref[0]
        coef = jnp.concatenate([cf, cs], axis=0)
        out_ref[...] = coef[:, :, None] * xT[None, :, :]

    in_specs = [
        pl.BlockSpec((D, D), lambda i: (0, 0)),
        pl.BlockSpec((D, tile), lambda i: (0, i)),
        pl.BlockSpec((T_FAST, D), lambda i: (0, 0)),
        pl.BlockSpec((T_SLOW, D), lambda i: (0, 0)),
        pl.BlockSpec(memory_space=pltpu.SMEM),
        pl.BlockSpec(memory_space=pltpu.SMEM),
    ]
    args = [W, eT, fast_basis, slow_basis,
            drive_scale.reshape(1), slow_scale.reshape(1)]
    aliases = {}
    if prev is not None:
        in_specs.append(pl.BlockSpec(memory_space=pl.ANY))
        args.append(prev)
        aliases = {6: 0}
    return pl.pallas_call(
        body,
        grid=(n // tile,),
        in_specs=in_specs,
        out_specs=pl.BlockSpec((T_ALL, D, tile),
                               lambda i: (0, 0, i + blk0)),
        out_shape=jax.ShapeDtypeStruct((T_ALL, D, n_total), jnp.float32),
        input_output_aliases=aliases,
    )(*args)


def kernel(token_ids, embed, W, drive_scale, fast_basis, slow_basis, slow_scale):
    n = token_ids.size
    ids = token_ids.reshape(-1).astype(jnp.int32)
    tT = jnp.swapaxes(embed, 0, 1)
    z = tT.reshape(RT, 8, CT, 128).transpose(0, 2, 1, 3)
    tile = 512
    n_chunk = 1
    nh = n // n_chunk
    outT = None
    for chunk in range(n_chunk):
        ids_h = lax.slice_in_dim(ids, chunk * nh, (chunk + 1) * nh)
        e4 = _sc_gather_cols(ids_h, z, nh)
        eT = e4.transpose(0, 2, 1, 3).reshape(D, nh)
        outT = _tc_expand_t(eT, W, fast_basis, slow_basis, drive_scale,
                            slow_scale, tile, n, chunk * (nh // tile),
                            prev=outT)
    return jnp.swapaxes(outT, 1, 2)
